# Initial kernel scaffold; baseline (speedup 1.0000x reference)
import jax
import jax.numpy as jnp
from jax.experimental import pallas as pl

MIN_ROUTE_LEN = 2
MAX_ROUTE_LEN = 16


def _cost_kernel(dem_ref, tt_ref, nt_ref, hp_ref, routes_ref, nrl_ref, hcr_ref,
                 trip_out_ref, scalars_ref, nsv_ref):
    dem = dem_ref[0]
    tt = tt_ref[0]
    nt = nt_ref[0]
    hp = hp_ref[0]

    nopath = jnp.logical_not(hp)
    trip_times = jnp.where(nopath, 0.0, tt)
    trip_out_ref[0] = trip_times

    nt_eff = jnp.where(nopath, 3, nt)
    total_dmd_time = jnp.sum(dem * trip_times)
    tat0 = jnp.sum(dem * (nt_eff == 0))
    tat1 = jnp.sum(dem * (nt_eff == 1))
    tat2 = jnp.sum(dem * (nt_eff == 2))
    d_un = jnp.sum(dem * (nt_eff > 2))
    total_demand = jnp.sum(dem)
    unserved = jnp.sum(dem * nopath)
    total_transfers = jnp.sum(dem * nt.astype(jnp.float32))

    s = routes_ref[0]
    R, L = s.shape
    valid = s > -1
    count = jnp.zeros((R,), jnp.float32)
    for i in range(L):
        si = s[:, i][:, None]
        before = jnp.arange(L)[None, :] < i
        dup = jnp.any((s == si) & before & valid, axis=1)
        first_occ = valid[:, i] & jnp.logical_not(dup)
        count = count + first_occ.astype(jnp.float32)
    nsv_ref[0, 0, :] = count

    route_lens = valid.sum(axis=1)
    delta = jnp.maximum(MIN_ROUTE_LEN - route_lens, 0)
    delta = jnp.where(route_lens == 0, 0, delta)
    delta = delta + jnp.maximum(route_lens - MAX_ROUTE_LEN, 0)
    n_unstarted = nrl_ref[0, 0, 0] - hcr_ref[0, 0, 0]
    n_stops_oob = delta.sum().astype(jnp.float32) + n_unstarted * MIN_ROUTE_LEN

    vec = jnp.stack([total_dmd_time, tat0, tat1, tat2, d_un,
                     total_demand, unserved, total_transfers, n_stops_oob])
    scalars_ref[0, 0, :] = vec


@jax.jit
def _run(demand, transit_times, n_transfers, has_path, batch_routes,
         nrl, hcr):
    B, N, _ = demand.shape
    _, R, L = batch_routes.shape
    grid = (B,)
    bs_full = pl.BlockSpec((1, N, N), lambda b: (b, 0, 0))
    bs_routes = pl.BlockSpec((1, R, L), lambda b: (b, 0, 0))
    bs_scalar = pl.BlockSpec((1, 1, 1), lambda b: (b, 0, 0))
    trip_times, scalars, nsv = pl.pallas_call(
        _cost_kernel,
        grid=grid,
        in_specs=[bs_full, bs_full, bs_full, bs_full, bs_routes,
                  bs_scalar, bs_scalar],
        out_specs=[bs_full,
                   pl.BlockSpec((1, 1, 9), lambda b: (b, 0, 0)),
                   pl.BlockSpec((1, 1, R), lambda b: (b, 0, 0))],
        out_shape=[jax.ShapeDtypeStruct((B, N, N), jnp.float32),
                   jax.ShapeDtypeStruct((B, 1, 9), jnp.float32),
                   jax.ShapeDtypeStruct((B, 1, R), jnp.float32)],
    )(demand, transit_times, n_transfers, has_path, batch_routes, nrl, hcr)
    return trip_times, scalars, nsv


def kernel(demand, transit_times, total_route_time, n_routes_left_to_plan,
           n_transfers, has_path, batch_routes, has_current_route,
           n_disconnected):
    B = demand.shape[0]
    R = batch_routes.shape[1]
    nrl = n_routes_left_to_plan.reshape(B, 1, 1)
    hcr = has_current_route.astype(jnp.float32).reshape(B, 1, 1)
    trip_times, scalars, nsv = _run(
        demand, transit_times, n_transfers, has_path, batch_routes, nrl, hcr)
    sc = scalars.reshape(B, 9)
    total_dmd_time = sc[:, 0]
    trips_at_transfers = sc[:, 1:5]
    total_demand = sc[:, 5]
    unserved_demand = sc[:, 6]
    total_transfers = sc[:, 7]
    n_stops_oob = sc[:, 8]
    n_stops_visited = nsv.reshape(B, R)
    return (total_dmd_time, total_route_time, trips_at_transfers,
            total_demand, unserved_demand, total_transfers, trip_times,
            n_disconnected, n_stops_oob, n_stops_visited)


# speedup vs baseline: 1.2856x; 1.2856x over previous
import jax
import jax.numpy as jnp
from jax import lax
from jax.experimental import pallas as pl
from jax.experimental.pallas import tpu as pltpu
from jax.experimental.pallas import tpu_sc as plsc

MIN_ROUTE_LEN = 2
MAX_ROUTE_LEN = 16


def _dense_kernel(dem_ref, tt_ref, nt_ref, hp_ref, trip_out_ref, scalars_ref):
    dem = dem_ref[0]
    tt = tt_ref[0]
    nt = nt_ref[0]
    hp = hp_ref[0]

    hpf = hp.astype(jnp.float32)
    trip_times = tt * hpf
    trip_out_ref[0] = trip_times

    zero = jnp.zeros((), jnp.float32)
    sd = dem * hpf
    total_dmd_time = jnp.sum(dem * trip_times)
    total_demand = jnp.sum(dem)
    served = jnp.sum(sd)
    unserved = total_demand - served
    total_transfers = jnp.sum(dem * nt.astype(jnp.float32))
    tat0 = jnp.sum(jnp.where(nt == 0, sd, zero))
    tat1 = jnp.sum(jnp.where(nt == 1, sd, zero))
    tat2 = jnp.sum(jnp.where(nt == 2, sd, zero))
    d_un = total_demand - tat0 - tat1 - tat2

    vec = jnp.stack([total_dmd_time, tat0, tat1, tat2, d_un,
                     total_demand, unserved, total_transfers])
    scalars_ref[0, 0, :] = vec


def _routes_sc_kernel(routes_hbm, nrl_hbm, hcr_hbm, nsv_hbm, oob_hbm,
                      routes_v, pos_v, counts_v, nrl_v, hcr_v, oob_v):
    R, L = routes_v.shape
    c = lax.axis_index("c")
    s = lax.axis_index("s")
    b = s * 2 + c

    pltpu.sync_copy(routes_hbm.at[b], routes_v)
    pltpu.sync_copy(nrl_hbm, nrl_v)
    pltpu.sync_copy(hcr_hbm, hcr_v)

    lanes = lax.iota(jnp.int32, 16)
    accs = [jnp.zeros((16,), jnp.float32) for _ in range(R // 16)]
    oob_acc = jnp.zeros((16,), jnp.float32)
    for r in range(R):
        idx = routes_v[r, :]
        valid = idx > -1
        safe = jnp.where(valid, idx, 0)
        plsc.store_scatter(pos_v, [safe], lanes, mask=valid)
        g = plsc.load_gather(pos_v, [safe], mask=valid)
        first = jnp.logical_and(g == lanes, valid)
        cnt = plsc.all_reduce_population_count(first).astype(jnp.float32)
        rlen = plsc.all_reduce_population_count(valid)
        delta = jnp.maximum(MIN_ROUTE_LEN - rlen, 0)
        delta = jnp.where(rlen == 0, 0, delta)
        delta = delta + jnp.maximum(rlen - MAX_ROUTE_LEN, 0)
        oob_acc = oob_acc + delta.astype(jnp.float32)
        sel = lanes == (r % 16)
        k = r // 16
        accs[k] = jnp.where(sel, cnt, accs[k])
    for k in range(R // 16):
        counts_v[pl.ds(k * 16, 16)] = accs[k]
    pltpu.sync_copy(counts_v, nsv_hbm.at[b])

    bvec = jnp.full((16,), 0, jnp.int32) + b
    nrlb = plsc.load_gather(nrl_v, [bvec])
    hcrb = plsc.load_gather(hcr_v, [bvec])
    oob_v[...] = oob_acc + (nrlb - hcrb) * float(MIN_ROUTE_LEN)
    pltpu.sync_copy(oob_v, oob_hbm.at[b])


@jax.jit
def _run(demand, transit_times, n_transfers, has_path, batch_routes,
         nrl, hcr):
    B, N, _ = demand.shape
    _, R, L = batch_routes.shape
    bs_full = pl.BlockSpec((1, N, N), lambda b: (b, 0, 0))
    trip_times, scalars = pl.pallas_call(
        _dense_kernel,
        grid=(B,),
        in_specs=[bs_full, bs_full, bs_full, bs_full],
        out_specs=[bs_full, pl.BlockSpec((1, 1, 8), lambda b: (b, 0, 0))],
        out_shape=[jax.ShapeDtypeStruct((B, N, N), jnp.float32),
                   jax.ShapeDtypeStruct((B, 1, 8), jnp.float32)],
    )(demand, transit_times, n_transfers, has_path)

    mesh = plsc.VectorSubcoreMesh(core_axis_name="c", subcore_axis_name="s",
                                  num_cores=2, num_subcores=16)
    nsv, oob = pl.kernel(
        _routes_sc_kernel,
        out_type=[jax.ShapeDtypeStruct((B, R), jnp.float32),
                  jax.ShapeDtypeStruct((B, 16), jnp.float32)],
        mesh=mesh,
        scratch_types=[pltpu.VMEM((R, L), jnp.int32),
                       pltpu.VMEM((N,), jnp.int32),
                       pltpu.VMEM((R,), jnp.float32),
                       pltpu.VMEM((B,), jnp.float32),
                       pltpu.VMEM((B,), jnp.float32),
                       pltpu.VMEM((16,), jnp.float32)],
        compiler_params=pltpu.CompilerParams(needs_layout_passes=False),
    )(batch_routes, nrl, hcr)
    return trip_times, scalars, nsv, oob


def kernel(demand, transit_times, total_route_time, n_routes_left_to_plan,
           n_transfers, has_path, batch_routes, has_current_route,
           n_disconnected):
    B = demand.shape[0]
    R = batch_routes.shape[1]
    hcr = has_current_route.astype(jnp.float32)
    trip_times, scalars, nsv, oob = _run(
        demand, transit_times, n_transfers, has_path, batch_routes,
        n_routes_left_to_plan, hcr)
    sc = scalars.reshape(B, 8)
    total_dmd_time = sc[:, 0]
    trips_at_transfers = sc[:, 1:5]
    total_demand = sc[:, 5]
    unserved_demand = sc[:, 6]
    total_transfers = sc[:, 7]
    n_stops_oob = oob[:, 0]
    n_stops_visited = nsv
    return (total_dmd_time, total_route_time, trips_at_transfers,
            total_demand, unserved_demand, total_transfers, trip_times,
            n_disconnected, n_stops_oob, n_stops_visited)
